# initial kernel scaffold (unmeasured)
import jax
import jax.numpy as jnp
from jax import lax
from jax.experimental import pallas as pl
from jax.experimental.pallas import tpu as pltpu

N_DEV = 4
S = 1024
H = 8
D = 128
HD = H * D
BLK = 64
BPS = S // BLK
SCALE = 0.08838834764831843


def kernel(x, Wq, K_ext, V_ext, Wo):
    x2 = x.reshape(S, HD)
    K2 = K_ext.reshape(S, HD)
    V2 = V_ext.reshape(S, HD)

    def body(x_ref, wq_ref, k_ref, v_ref, wo_ref, out_ref,
             kbuf, vbuf, ksend, krecv, vsend, vrecv):
        my = lax.axis_index("i")
        left = lax.rem(my + N_DEV - 1, N_DEV)
        right = lax.rem(my + 1, N_DEV)

        barrier = pltpu.get_barrier_semaphore()
        for nbr in (left, right):
            pl.semaphore_signal(barrier, inc=1, device_id=(nbr,),
                                device_id_type=pl.DeviceIdType.MESH)
        pl.semaphore_wait(barrier, 2)

        Q = jnp.dot(x_ref[...], wq_ref[...],
                    preferred_element_type=jnp.float32)

        rb = lax.broadcasted_iota(jnp.int32, (S, 1), 0) // BLK
        cb = lax.broadcasted_iota(jnp.int32, (1, S), 1) // BLK

        acc = [jnp.zeros((S, D), jnp.float32) for _ in range(H)]
        den = [jnp.zeros((S, 1), jnp.float32) for _ in range(H)]

        def process(kc, vc, origin):
            mask = (my * BPS + rb) >= (origin * BPS + cb)
            for h in range(H):
                qh = Q[:, h * D:(h + 1) * D]
                kh = kc[:, h * D:(h + 1) * D]
                s = lax.dot_general(
                    qh, kh, (((1,), (1,)), ((), ())),
                    preferred_element_type=jnp.float32) * SCALE
                w = jnp.where(mask, jnp.exp(s), 0.0)
                den[h] = den[h] + jnp.sum(w, axis=1, keepdims=True)
                acc[h] = acc[h] + jnp.dot(
                    w, vc[:, h * D:(h + 1) * D],
                    preferred_element_type=jnp.float32)

        process(k_ref[...], v_ref[...], my)

        for hop in range(1, N_DEV):
            s_slot = (hop - 1) % 2
            r_slot = hop % 2
            ksrc = k_ref if hop == 1 else kbuf.at[s_slot]
            vsrc = v_ref if hop == 1 else vbuf.at[s_slot]
            k_rdma = pltpu.make_async_remote_copy(
                src_ref=ksrc, dst_ref=kbuf.at[r_slot],
                send_sem=ksend.at[hop - 1], recv_sem=krecv.at[hop - 1],
                device_id=(right,), device_id_type=pl.DeviceIdType.MESH)
            v_rdma = pltpu.make_async_remote_copy(
                src_ref=vsrc, dst_ref=vbuf.at[r_slot],
                send_sem=vsend.at[hop - 1], recv_sem=vrecv.at[hop - 1],
                device_id=(right,), device_id_type=pl.DeviceIdType.MESH)
            k_rdma.start()
            v_rdma.start()
            k_rdma.wait()
            v_rdma.wait()
            origin = lax.rem(my - hop + N_DEV, N_DEV)
            process(kbuf[r_slot], vbuf[r_slot], origin)

        ctx = jnp.concatenate([acc[h] / den[h] for h in range(H)], axis=1)
        out_ref[...] = jnp.dot(ctx, wo_ref[...],
                               preferred_element_type=jnp.float32)

    out = pl.pallas_call(
        body,
        out_shape=jax.ShapeDtypeStruct((S, HD), jnp.float32),
        in_specs=[pl.BlockSpec(memory_space=pltpu.VMEM)] * 5,
        out_specs=pl.BlockSpec(memory_space=pltpu.VMEM),
        scratch_shapes=[
            pltpu.VMEM((2, S, HD), jnp.float32),
            pltpu.VMEM((2, S, HD), jnp.float32),
            pltpu.SemaphoreType.DMA((N_DEV - 1,)),
            pltpu.SemaphoreType.DMA((N_DEV - 1,)),
            pltpu.SemaphoreType.DMA((N_DEV - 1,)),
            pltpu.SemaphoreType.DMA((N_DEV - 1,)),
        ],
        compiler_params=pltpu.CompilerParams(collective_id=0),
    )(x2, Wq, K2, V2, Wo)
    return out.reshape(1, S, HD)


# baseline (device time: 347380 ns/iter reference)
import jax
import jax.numpy as jnp
from jax import lax
from jax.experimental import pallas as pl
from jax.experimental.pallas import tpu as pltpu

N_DEV = 4
S = 1024
H = 8
D = 128
HD = H * D
BLK = 64
BPS = S // BLK
SCALE = 0.08838834764831843


def kernel(x, Wq, K_ext, V_ext, Wo):
    x2 = x.reshape(S, HD)
    K2 = K_ext.reshape(S, HD)
    V2 = V_ext.reshape(S, HD)

    def body(x_ref, wq_ref, k_ref, v_ref, wo_ref, out_ref,
             kbuf, vbuf, q_ref, ctx_ref, den_ref,
             ksend, krecv, vsend, vrecv):
        my = lax.axis_index("i")
        left = lax.rem(my + N_DEV - 1, N_DEV)
        right = lax.rem(my + 1, N_DEV)

        barrier = pltpu.get_barrier_semaphore()
        for nbr in (left, right):
            pl.semaphore_signal(barrier, inc=1, device_id=(nbr,),
                                device_id_type=pl.DeviceIdType.MESH)
        pl.semaphore_wait(barrier, 2)

        q_ref[...] = jnp.dot(x_ref[...], wq_ref[...],
                             preferred_element_type=jnp.float32)
        ctx_ref[...] = jnp.zeros((S, HD), jnp.float32)
        den_ref[...] = jnp.zeros((S, H), jnp.float32)

        rb = lax.broadcasted_iota(jnp.int32, (S, 1), 0) // BLK
        cb = lax.broadcasted_iota(jnp.int32, (1, S), 1) // BLK

        def process(kc_ref, vc_ref, origin):
            mask = (my * BPS + rb) >= (origin * BPS + cb)
            for h in range(H):
                hs = slice(h * D, (h + 1) * D)
                s = lax.dot_general(
                    q_ref[:, hs], kc_ref[:, hs], (((1,), (1,)), ((), ())),
                    preferred_element_type=jnp.float32) * SCALE
                w = jnp.where(mask, jnp.exp(s), 0.0)
                den_ref[:, h:h + 1] = den_ref[:, h:h + 1] + jnp.sum(
                    w, axis=1, keepdims=True)
                ctx_ref[:, hs] = ctx_ref[:, hs] + jnp.dot(
                    w, vc_ref[:, hs], preferred_element_type=jnp.float32)

        process(k_ref, v_ref, my)

        for hop in range(1, N_DEV):
            s_slot = (hop - 1) % 2
            r_slot = hop % 2
            ksrc = k_ref if hop == 1 else kbuf.at[s_slot]
            vsrc = v_ref if hop == 1 else vbuf.at[s_slot]
            k_rdma = pltpu.make_async_remote_copy(
                src_ref=ksrc, dst_ref=kbuf.at[r_slot],
                send_sem=ksend.at[hop - 1], recv_sem=krecv.at[hop - 1],
                device_id=(right,), device_id_type=pl.DeviceIdType.MESH)
            v_rdma = pltpu.make_async_remote_copy(
                src_ref=vsrc, dst_ref=vbuf.at[r_slot],
                send_sem=vsend.at[hop - 1], recv_sem=vrecv.at[hop - 1],
                device_id=(right,), device_id_type=pl.DeviceIdType.MESH)
            k_rdma.start()
            v_rdma.start()
            k_rdma.wait()
            v_rdma.wait()
            origin = lax.rem(my - hop + N_DEV, N_DEV)
            process(kbuf.at[r_slot], vbuf.at[r_slot], origin)

        for h in range(H):
            hs = slice(h * D, (h + 1) * D)
            ctx_ref[:, hs] = ctx_ref[:, hs] / den_ref[:, h:h + 1]
        out_ref[...] = jnp.dot(ctx_ref[...], wo_ref[...],
                               preferred_element_type=jnp.float32)

    out = pl.pallas_call(
        body,
        out_shape=jax.ShapeDtypeStruct((S, HD), jnp.float32),
        in_specs=[pl.BlockSpec(memory_space=pltpu.VMEM)] * 5,
        out_specs=pl.BlockSpec(memory_space=pltpu.VMEM),
        scratch_shapes=[
            pltpu.VMEM((2, S, HD), jnp.float32),
            pltpu.VMEM((2, S, HD), jnp.float32),
            pltpu.VMEM((S, HD), jnp.float32),
            pltpu.VMEM((S, HD), jnp.float32),
            pltpu.VMEM((S, H), jnp.float32),
            pltpu.SemaphoreType.DMA((N_DEV - 1,)),
            pltpu.SemaphoreType.DMA((N_DEV - 1,)),
            pltpu.SemaphoreType.DMA((N_DEV - 1,)),
            pltpu.SemaphoreType.DMA((N_DEV - 1,)),
        ],
        compiler_params=pltpu.CompilerParams(
            collective_id=0,
            vmem_limit_bytes=63 * 1024 * 1024,
        ),
    )(x2, Wq, K2, V2, Wo)
    return out.reshape(1, S, HD)


# device time: 177213 ns/iter; 1.9602x vs baseline; 1.9602x over previous
import jax
import jax.numpy as jnp
from jax import lax
from jax.experimental import pallas as pl
from jax.experimental.pallas import tpu as pltpu

N_DEV = 4
S = 1024
H = 8
D = 128
HD = H * D
BLK = 64
BPS = S // BLK
SCALE = 0.08838834764831843


def kernel(x, Wq, K_ext, V_ext, Wo):
    x2 = x.reshape(S, HD)
    K2 = K_ext.reshape(S, HD).astype(jnp.bfloat16)
    V2 = V_ext.reshape(S, HD).astype(jnp.bfloat16)

    def body(x_ref, wq_ref, k_ref, v_ref, wo_ref, out_ref,
             kbuf, vbuf, q_ref, ctx_ref, den_ref,
             ksend, krecv, vsend, vrecv):
        my = lax.axis_index("i")
        left = lax.rem(my + N_DEV - 1, N_DEV)
        right = lax.rem(my + 1, N_DEV)

        barrier = pltpu.get_barrier_semaphore()
        for nbr in (left, right):
            pl.semaphore_signal(barrier, inc=1, device_id=(nbr,),
                                device_id_type=pl.DeviceIdType.MESH)
        pl.semaphore_wait(barrier, 2)

        def make_rdma(hop, ksrc, vsrc):
            k_rdma = pltpu.make_async_remote_copy(
                src_ref=ksrc, dst_ref=kbuf.at[hop - 1],
                send_sem=ksend.at[hop - 1], recv_sem=krecv.at[hop - 1],
                device_id=(right,), device_id_type=pl.DeviceIdType.MESH)
            v_rdma = pltpu.make_async_remote_copy(
                src_ref=vsrc, dst_ref=vbuf.at[hop - 1],
                send_sem=vsend.at[hop - 1], recv_sem=vrecv.at[hop - 1],
                device_id=(right,), device_id_type=pl.DeviceIdType.MESH)
            k_rdma.start()
            v_rdma.start()
            return k_rdma, v_rdma

        rb = lax.broadcasted_iota(jnp.int32, (S, 1), 0) // BLK
        cb = lax.broadcasted_iota(jnp.int32, (1, S), 1) // BLK

        def process(kc_ref, vc_ref, origin):
            mask = (my * BPS + rb) >= (origin * BPS + cb)
            for h in range(H):
                hs = slice(h * D, (h + 1) * D)
                s = lax.dot_general(
                    q_ref[:, hs], kc_ref[:, hs], (((1,), (1,)), ((), ())),
                    preferred_element_type=jnp.float32) * SCALE
                w = jnp.where(mask, jnp.exp(s), 0.0)
                den_ref[:, h:h + 1] = den_ref[:, h:h + 1] + jnp.sum(
                    w, axis=1, keepdims=True)
                ctx_ref[:, hs] = ctx_ref[:, hs] + jnp.dot(
                    w.astype(jnp.bfloat16), vc_ref[:, hs],
                    preferred_element_type=jnp.float32)

        rdmas = make_rdma(1, k_ref, v_ref)
        q_ref[...] = jnp.dot(x_ref[...], wq_ref[...],
                             preferred_element_type=jnp.float32
                             ).astype(jnp.bfloat16)
        ctx_ref[...] = jnp.zeros((S, HD), jnp.float32)
        den_ref[...] = jnp.zeros((S, H), jnp.float32)
        process(k_ref, v_ref, my)

        for hop in range(1, N_DEV):
            for r in rdmas:
                r.wait()
            slot = hop - 1
            if hop < N_DEV - 1:
                rdmas = make_rdma(hop + 1, kbuf.at[slot], vbuf.at[slot])
            origin = lax.rem(my - hop + N_DEV, N_DEV)
            process(kbuf.at[slot], vbuf.at[slot], origin)

        for h in range(H):
            hs = slice(h * D, (h + 1) * D)
            ctx_ref[:, hs] = ctx_ref[:, hs] / den_ref[:, h:h + 1]
        out_ref[...] = jnp.dot(ctx_ref[...], wo_ref[...],
                               preferred_element_type=jnp.float32)

    out = pl.pallas_call(
        body,
        out_shape=jax.ShapeDtypeStruct((S, HD), jnp.float32),
        in_specs=[pl.BlockSpec(memory_space=pltpu.VMEM)] * 5,
        out_specs=pl.BlockSpec(memory_space=pltpu.VMEM),
        scratch_shapes=[
            pltpu.VMEM((N_DEV - 1, S, HD), jnp.bfloat16),
            pltpu.VMEM((N_DEV - 1, S, HD), jnp.bfloat16),
            pltpu.VMEM((S, HD), jnp.bfloat16),
            pltpu.VMEM((S, HD), jnp.float32),
            pltpu.VMEM((S, H), jnp.float32),
            pltpu.SemaphoreType.DMA((N_DEV - 1,)),
            pltpu.SemaphoreType.DMA((N_DEV - 1,)),
            pltpu.SemaphoreType.DMA((N_DEV - 1,)),
            pltpu.SemaphoreType.DMA((N_DEV - 1,)),
        ],
        compiler_params=pltpu.CompilerParams(
            collective_id=0,
            vmem_limit_bytes=63 * 1024 * 1024,
        ),
    )(x2, Wq, K2, V2, Wo)
    return out.reshape(1, S, HD)


# device time: 105952 ns/iter; 3.2787x vs baseline; 1.6726x over previous
import jax
import jax.numpy as jnp
from jax import lax
from jax.experimental import pallas as pl
from jax.experimental.pallas import tpu as pltpu

N_DEV = 4
S = 1024
H = 8
D = 128
HD = H * D
BLK = 64
BPS = S // BLK
SCALE = 0.08838834764831843
HALF = S // 2


def kernel(x, Wq, K_ext, V_ext, Wo):
    x2 = x.reshape(S, HD)
    K2 = K_ext.reshape(S, HD).astype(jnp.bfloat16)
    V2 = V_ext.reshape(S, HD).astype(jnp.bfloat16)

    def body(x_ref, wq_ref, k_ref, v_ref, wo_ref, out_ref,
             kR, vR, kL, vL, kH, vH, q_ref, ctx_ref, den_ref,
             send_sems, recv_sems):
        my = lax.axis_index("i")
        left = lax.rem(my + N_DEV - 1, N_DEV)
        right = lax.rem(my + 1, N_DEV)

        barrier = pltpu.get_barrier_semaphore()
        for nbr in (left, right):
            pl.semaphore_signal(barrier, inc=1, device_id=(nbr,),
                                device_id_type=pl.DeviceIdType.MESH)
        pl.semaphore_wait(barrier, 2)

        def rdma(i, src, dst, dev):
            r = pltpu.make_async_remote_copy(
                src_ref=src, dst_ref=dst,
                send_sem=send_sems.at[i], recv_sem=recv_sems.at[i],
                device_id=(dev,), device_id_type=pl.DeviceIdType.MESH)
            r.start()
            return r

        hop1 = [
            rdma(0, k_ref, kR, right), rdma(1, v_ref, vR, right),
            rdma(2, k_ref, kL, left), rdma(3, v_ref, vL, left),
        ]

        rb = lax.broadcasted_iota(jnp.int32, (S, 1), 0) // BLK
        cb = lax.broadcasted_iota(jnp.int32, (1, S), 1) // BLK

        def attend(kc_ref, vc_ref, masked):
            for h in range(H):
                hs = slice(h * D, (h + 1) * D)
                s = lax.dot_general(
                    q_ref[:, hs], kc_ref[:, hs], (((1,), (1,)), ((), ())),
                    preferred_element_type=jnp.float32) * SCALE
                w = jnp.exp(s)
                if masked:
                    w = jnp.where(rb >= cb, w, 0.0)
                den_ref[:, h:h + 1] = den_ref[:, h:h + 1] + jnp.sum(
                    w, axis=1, keepdims=True)
                ctx_ref[:, hs] = ctx_ref[:, hs] + jnp.dot(
                    w.astype(jnp.bfloat16), vc_ref[:, hs],
                    preferred_element_type=jnp.float32)

        def process(kc_ref, vc_ref, origin):
            @pl.when(origin < my)
            def _():
                attend(kc_ref, vc_ref, masked=False)

        q_ref[...] = jnp.dot(x_ref[...], wq_ref[...],
                             preferred_element_type=jnp.float32
                             ).astype(jnp.bfloat16)
        ctx_ref[...] = jnp.zeros((S, HD), jnp.float32)
        den_ref[...] = jnp.zeros((S, H), jnp.float32)
        attend(k_ref, v_ref, masked=True)

        for r in hop1:
            r.wait()

        lo = (pl.ds(0, HALF), slice(None))
        hi = (pl.ds(HALF, HALF), slice(None))
        hop2 = [
            rdma(4, kR.at[lo], kH.at[lo], right),
            rdma(5, vR.at[lo], vH.at[lo], right),
            rdma(6, kL.at[hi], kH.at[hi], left),
            rdma(7, vL.at[hi], vH.at[hi], left),
        ]

        process(kR, vR, left)
        process(kL, vL, right)

        for r in hop2:
            r.wait()
        process(kH, vH, lax.rem(my + 2, N_DEV))

        for h in range(H):
            hs = slice(h * D, (h + 1) * D)
            ctx_ref[:, hs] = ctx_ref[:, hs] / den_ref[:, h:h + 1]
        out_ref[...] = jnp.dot(ctx_ref[...], wo_ref[...],
                               preferred_element_type=jnp.float32)

    out = pl.pallas_call(
        body,
        out_shape=jax.ShapeDtypeStruct((S, HD), jnp.float32),
        in_specs=[pl.BlockSpec(memory_space=pltpu.VMEM)] * 5,
        out_specs=pl.BlockSpec(memory_space=pltpu.VMEM),
        scratch_shapes=[
            pltpu.VMEM((S, HD), jnp.bfloat16),
            pltpu.VMEM((S, HD), jnp.bfloat16),
            pltpu.VMEM((S, HD), jnp.bfloat16),
            pltpu.VMEM((S, HD), jnp.bfloat16),
            pltpu.VMEM((S, HD), jnp.bfloat16),
            pltpu.VMEM((S, HD), jnp.bfloat16),
            pltpu.VMEM((S, HD), jnp.bfloat16),
            pltpu.VMEM((S, HD), jnp.float32),
            pltpu.VMEM((S, H), jnp.float32),
            pltpu.SemaphoreType.DMA((8,)),
            pltpu.SemaphoreType.DMA((8,)),
        ],
        compiler_params=pltpu.CompilerParams(
            collective_id=0,
            vmem_limit_bytes=63 * 1024 * 1024,
        ),
    )(x2, Wq, K2, V2, Wo)
    return out.reshape(1, S, HD)
